# baseline (speedup 1.0000x reference)
import jax
import jax.numpy as jnp
from jax.experimental import pallas as pl
from jax.experimental.pallas import tpu as pltpu
from jax.experimental.pallas import tpu_sc as plsc

N_TOK = 65536
N_EMB = 1024
DIM = 64
BLK = 2048
NCHUNK = 2
CHT = N_TOK // NCHUNK

SC_NC = 2
SC_NS = 16
SC_NW = SC_NC * SC_NS
SC_CH = 128

SL_BLK = 4096


def _tc_body(x_ref, w_ref, idxe_ref, idxo_ref, loss_ref, acc_ref):
    i = pl.program_id(0)
    x2 = x_ref[...]
    w = w_ref[...]
    b = jnp.sum(w * w, axis=1)[None, :]

    @pl.when(i == 0)
    def _():
        acc_ref[0] = 0.0

    tot = jnp.float32(0.0)
    for lo, idx_ref in ((0, idxe_ref), (DIM, idxo_ref)):
        xp = x2[:, lo:lo + DIM]
        c = jax.lax.dot_general(xp, w, (((1,), (1,)), ((), ())),
                                preferred_element_type=jnp.float32)
        a = jnp.sum(xp * xp, axis=1, keepdims=True)
        dist = (a + b) - 2.0 * c
        m = jnp.min(dist, axis=1, keepdims=True)
        jidx = jax.lax.broadcasted_iota(
            jnp.int32, dist.shape, 1).astype(jnp.float32)
        idxf = jnp.min(jnp.where(dist == m, jidx, float(N_EMB)), axis=1)
        idx_ref[...] = idxf.astype(jnp.int32).reshape(BLK // 256, 128)
        tot = tot + jnp.sum(m)

    acc_ref[0] += tot

    @pl.when(i == pl.num_programs(0) - 1)
    def _():
        loss_ref[...] = jnp.full((1, 1), acc_ref[0], dtype=jnp.float32)


def _tc_argmin_loss(x128, weight, k):
    g = CHT // BLK
    pr = BLK // 256
    return pl.pallas_call(
        _tc_body,
        grid=(g,),
        in_specs=[
            pl.BlockSpec((BLK // 2, 128), lambda i, k=k: (i + k * g, 0)),
            pl.BlockSpec((N_EMB, DIM), lambda i: (0, 0)),
        ],
        out_specs=[
            pl.BlockSpec((pr, 128), lambda i: (i, 0)),
            pl.BlockSpec((pr, 128), lambda i: (i, 0)),
            pl.BlockSpec((1, 1), lambda i: (0, 0)),
        ],
        out_shape=[
            jax.ShapeDtypeStruct((CHT // 256, 128), jnp.int32),
            jax.ShapeDtypeStruct((CHT // 256, 128), jnp.int32),
            jax.ShapeDtypeStruct((1, 1), jnp.float32),
        ],
        scratch_shapes=[pltpu.SMEM((1,), jnp.float32)],
    )(x128, weight)


def _sc_gather(w_pad, idx):
    n = idx.shape[0]
    rows_per_w = n // SC_NW
    n_ch = rows_per_w // SC_CH
    mesh = plsc.VectorSubcoreMesh(core_axis_name="c", subcore_axis_name="s")

    @pl.kernel(out_type=jax.ShapeDtypeStruct((n, 128), jnp.float32),
               mesh=mesh,
               scratch_types=[
                   pltpu.VMEM((SC_CH,), jnp.int32),
                   pltpu.VMEM((SC_CH, 128), jnp.float32),
                   pltpu.SemaphoreType.DMA,
               ])
    def k(w_hbm, i_hbm, o_hbm, idx_v, rows_v, sem):
        wid = jax.lax.axis_index("s") * SC_NC + jax.lax.axis_index("c")
        base = wid * rows_per_w

        @pl.loop(0, n_ch)
        def _(c):
            off = base + c * SC_CH
            pltpu.sync_copy(i_hbm.at[pl.ds(off, SC_CH)], idx_v)
            pltpu.async_copy(w_hbm.at[idx_v], rows_v, sem).wait()
            pltpu.sync_copy(rows_v, o_hbm.at[pl.ds(off, SC_CH)])

    return k(w_pad, idx)


def _pack_body(_, qe_ref, qo_ref, o_ref):
    o_ref[...] = jnp.concatenate(
        [qe_ref[:, :DIM], qo_ref[:, :DIM]], axis=1)


def _pack_first(q_raw):
    g = (CHT // 2) // SL_BLK
    return pl.pallas_call(
        lambda qe, qo, o: _pack_body(None, qe, qo, o),
        grid=(g,),
        in_specs=[
            pl.BlockSpec((SL_BLK, 128), lambda i: (i, 0)),
            pl.BlockSpec((SL_BLK, 128), lambda i, g=g: (i + g, 0)),
        ],
        out_specs=pl.BlockSpec((SL_BLK, 128), lambda i: (i, 0)),
        out_shape=jax.ShapeDtypeStruct((N_TOK // 2, 128), jnp.float32),
    )(q_raw, q_raw)


def _pack_into(buf, q_raw, k):
    g = (CHT // 2) // SL_BLK
    return pl.pallas_call(
        _pack_body,
        grid=(g,),
        in_specs=[
            pl.BlockSpec(memory_space=pl.ANY),
            pl.BlockSpec((SL_BLK, 128), lambda i: (i, 0)),
            pl.BlockSpec((SL_BLK, 128), lambda i, g=g: (i + g, 0)),
        ],
        out_specs=pl.BlockSpec((SL_BLK, 128),
                               lambda i, k=k, g=g: (i + k * g, 0)),
        out_shape=jax.ShapeDtypeStruct((N_TOK // 2, 128), jnp.float32),
        input_output_aliases={0: 0},
    )(buf, q_raw, q_raw)


def kernel(inputs, weight):
    w_pad = jnp.concatenate(
        [weight, jnp.zeros((N_EMB, 128 - DIM), jnp.float32)], axis=1)
    x128 = inputs.reshape(N_TOK // 2, 128)
    idx_parts, loss_parts, q_raws = [], [], []
    for k in range(NCHUNK):
        idxe2d, idxo2d, lsum = _tc_argmin_loss(x128, weight, k)
        idx_e = idxe2d.reshape(CHT // 2)
        idx_o = idxo2d.reshape(CHT // 2)
        q_raws.append(_sc_gather(w_pad, jnp.concatenate([idx_e, idx_o])))
        idx_parts.append(jnp.stack([idx_e, idx_o], axis=1).reshape(CHT))
        loss_parts.append(lsum[0, 0])
    packed = _pack_first(q_raws[0])
    for k in range(1, NCHUNK):
        packed = _pack_into(packed, q_raws[k], k)
    quantized = packed.reshape(N_TOK, DIM)
    loss = sum(loss_parts) * (1.25 / (N_TOK * DIM))
    indices = jnp.concatenate(idx_parts, axis=0)
    return loss, quantized, indices


# speedup vs baseline: 1.1902x; 1.1902x over previous
import jax
import jax.numpy as jnp
from jax.experimental import pallas as pl
from jax.experimental.pallas import tpu as pltpu
from jax.experimental.pallas import tpu_sc as plsc

N_TOK = 65536
N_EMB = 1024
DIM = 64
BLK = 1024
NCHUNK = 2
CHT = N_TOK // NCHUNK

SC_NC = 2
SC_NS = 16
SC_NW = SC_NC * SC_NS
SC_CH = 128


def _tc_body(x_ref, w_ref, idx_ref, loss_ref, acc_ref):
    i = pl.program_id(0)
    x = x_ref[...]
    w = w_ref[...]
    c = jax.lax.dot_general(x, w, (((1,), (1,)), ((), ())),
                            preferred_element_type=jnp.float32)
    a = jnp.sum(x * x, axis=1, keepdims=True)
    b = jnp.sum(w * w, axis=1)[None, :]
    dist = (a + b) - 2.0 * c
    m = jnp.min(dist, axis=1, keepdims=True)
    jidx = jax.lax.broadcasted_iota(
        jnp.int32, dist.shape, 1).astype(jnp.float32)
    idxf = jnp.min(jnp.where(dist == m, jidx, float(N_EMB)), axis=1)
    idx_ref[...] = idxf.astype(jnp.int32).reshape(BLK // 128, 128)

    @pl.when(i == 0)
    def _():
        acc_ref[0] = 0.0

    acc_ref[0] += jnp.sum(m)

    @pl.when(i == pl.num_programs(0) - 1)
    def _():
        loss_ref[...] = jnp.full((1, 1), acc_ref[0], dtype=jnp.float32)


def _tc_argmin_loss(inputs, weight, k):
    g = CHT // BLK
    return pl.pallas_call(
        _tc_body,
        grid=(g,),
        in_specs=[
            pl.BlockSpec((BLK, DIM), lambda i, k=k: (i + k * g, 0)),
            pl.BlockSpec((N_EMB, DIM), lambda i: (0, 0)),
        ],
        out_specs=[
            pl.BlockSpec((BLK // 128, 128), lambda i: (i, 0)),
            pl.BlockSpec((1, 1), lambda i: (0, 0)),
        ],
        out_shape=[
            jax.ShapeDtypeStruct((CHT // 128, 128), jnp.int32),
            jax.ShapeDtypeStruct((1, 1), jnp.float32),
        ],
        scratch_shapes=[pltpu.SMEM((1,), jnp.float32)],
    )(inputs, weight)


def _sc_gather(w_pad, idx):
    n = idx.shape[0]
    rows_per_w = n // SC_NW
    n_ch = rows_per_w // SC_CH
    mesh = plsc.VectorSubcoreMesh(core_axis_name="c", subcore_axis_name="s")

    @pl.kernel(out_type=jax.ShapeDtypeStruct((n, 128), jnp.float32),
               mesh=mesh,
               scratch_types=[
                   pltpu.VMEM((rows_per_w,), jnp.int32),
                   pltpu.VMEM((SC_CH, 128), jnp.float32),
                   pltpu.VMEM((SC_CH, 128), jnp.float32),
                   pltpu.SemaphoreType.DMA,
                   pltpu.SemaphoreType.DMA,
                   pltpu.SemaphoreType.DMA,
                   pltpu.SemaphoreType.DMA,
               ])
    def k(w_hbm, i_hbm, o_hbm, idx_all, buf0, buf1, sg0, sg1, sw0, sw1):
        wid = jax.lax.axis_index("s") * SC_NC + jax.lax.axis_index("c")
        base = wid * rows_per_w
        pltpu.sync_copy(i_hbm.at[pl.ds(base, rows_per_w)], idx_all)
        bufs = (buf0, buf1)
        gsems = (sg0, sg1)
        wsems = (sw0, sw1)
        gathers = [None] * n_ch
        writes = [None] * n_ch
        for c in range(n_ch):
            s = c % 2
            if c >= 2:
                writes[c - 2].wait()
            gathers[c] = pltpu.async_copy(
                w_hbm.at[idx_all.at[pl.ds(c * SC_CH, SC_CH)]], bufs[s],
                gsems[s])
            if c >= 1:
                gathers[c - 1].wait()
                writes[c - 1] = pltpu.async_copy(
                    bufs[(c - 1) % 2],
                    o_hbm.at[pl.ds(base + (c - 1) * SC_CH, SC_CH)],
                    wsems[(c - 1) % 2])
        gathers[n_ch - 1].wait()
        writes[n_ch - 1] = pltpu.async_copy(
            bufs[(n_ch - 1) % 2],
            o_hbm.at[pl.ds(base + (n_ch - 1) * SC_CH, SC_CH)],
            wsems[(n_ch - 1) % 2])
        if n_ch >= 2:
            writes[n_ch - 2].wait()
        writes[n_ch - 1].wait()

    return k(w_pad, idx)


def kernel(inputs, weight):
    w_pad = jnp.concatenate(
        [weight, jnp.zeros((N_EMB, 128 - DIM), jnp.float32)], axis=1)
    quantized = jnp.zeros((N_TOK, DIM), jnp.float32)
    idx_parts, loss_parts = [], []
    for k in range(NCHUNK):
        idx2d, lsum = _tc_argmin_loss(inputs, weight, k)
        idx = idx2d.reshape(CHT)
        q_raw = _sc_gather(w_pad, idx)
        quantized = jax.lax.dynamic_update_slice(
            quantized, q_raw[:, :DIM], (k * CHT, 0))
        idx_parts.append(idx)
        loss_parts.append(lsum[0, 0])
    loss = sum(loss_parts) * (1.25 / (N_TOK * DIM))
    indices = jnp.concatenate(idx_parts, axis=0)
    return loss, quantized, indices


# speedup vs baseline: 1.4188x; 1.1921x over previous
import jax
import jax.numpy as jnp
from jax.experimental import pallas as pl
from jax.experimental.pallas import tpu as pltpu
from jax.experimental.pallas import tpu_sc as plsc

N_TOK = 65536
N_EMB = 1024
DIM = 64
BLK = 1024
NCHUNK = 2
CHT = N_TOK // NCHUNK

SC_NC = 2
SC_NS = 16
SC_NW = SC_NC * SC_NS
SC_CH = 128


def _tc_body(xt_ref, w_ref, idx_ref, loss_ref, acc_ref):
    i = pl.program_id(0)
    xt = xt_ref[...]
    w = w_ref[...]
    c = jax.lax.dot_general(w, xt, (((1,), (0,)), ((), ())),
                            preferred_element_type=jnp.float32)
    a = jnp.sum(xt * xt, axis=0, keepdims=True)
    b = jnp.sum(w * w, axis=1, keepdims=True)
    dist = (b + a) - 2.0 * c
    m = jnp.min(dist, axis=0, keepdims=True)
    jidx = jax.lax.broadcasted_iota(
        jnp.int32, dist.shape, 0).astype(jnp.float32)
    idxf = jnp.min(jnp.where(dist == m, jidx, float(N_EMB)), axis=0)
    idx_ref[...] = idxf.astype(jnp.int32).reshape(1, 1, BLK)

    @pl.when(i == 0)
    def _():
        acc_ref[0] = 0.0

    acc_ref[0] += jnp.sum(m)

    @pl.when(i == pl.num_programs(0) - 1)
    def _():
        loss_ref[...] = jnp.full((1, 1), acc_ref[0], dtype=jnp.float32)


def _tc_argmin_loss(xt, weight, k):
    g = CHT // BLK
    return pl.pallas_call(
        _tc_body,
        grid=(g,),
        in_specs=[
            pl.BlockSpec((DIM, BLK), lambda i, k=k: (0, i + k * g)),
            pl.BlockSpec((N_EMB, DIM), lambda i: (0, 0)),
        ],
        out_specs=[
            pl.BlockSpec((1, 1, BLK), lambda i: (i, 0, 0)),
            pl.BlockSpec((1, 1), lambda i: (0, 0)),
        ],
        out_shape=[
            jax.ShapeDtypeStruct((g, 1, BLK), jnp.int32),
            jax.ShapeDtypeStruct((1, 1), jnp.float32),
        ],
        scratch_shapes=[pltpu.SMEM((1,), jnp.float32)],
    )(xt, weight)


def _sc_gather(w_pad, idx):
    n = idx.shape[0]
    rows_per_w = n // SC_NW
    n_ch = rows_per_w // SC_CH
    mesh = plsc.VectorSubcoreMesh(core_axis_name="c", subcore_axis_name="s")

    @pl.kernel(out_type=jax.ShapeDtypeStruct((n, 128), jnp.float32),
               mesh=mesh,
               scratch_types=[
                   pltpu.VMEM((rows_per_w,), jnp.int32),
                   pltpu.VMEM((SC_CH, 128), jnp.float32),
                   pltpu.VMEM((SC_CH, 128), jnp.float32),
                   pltpu.SemaphoreType.DMA,
                   pltpu.SemaphoreType.DMA,
                   pltpu.SemaphoreType.DMA,
                   pltpu.SemaphoreType.DMA,
               ])
    def k(w_hbm, i_hbm, o_hbm, idx_all, buf0, buf1, sg0, sg1, sw0, sw1):
        wid = jax.lax.axis_index("s") * SC_NC + jax.lax.axis_index("c")
        base = wid * rows_per_w
        pltpu.sync_copy(i_hbm.at[pl.ds(base, rows_per_w)], idx_all)
        bufs = (buf0, buf1)
        gsems = (sg0, sg1)
        wsems = (sw0, sw1)
        gathers = [None] * n_ch
        writes = [None] * n_ch
        for c in range(n_ch):
            s = c % 2
            if c >= 2:
                writes[c - 2].wait()
            gathers[c] = pltpu.async_copy(
                w_hbm.at[idx_all.at[pl.ds(c * SC_CH, SC_CH)]], bufs[s],
                gsems[s])
            if c >= 1:
                gathers[c - 1].wait()
                writes[c - 1] = pltpu.async_copy(
                    bufs[(c - 1) % 2],
                    o_hbm.at[pl.ds(base + (c - 1) * SC_CH, SC_CH)],
                    wsems[(c - 1) % 2])
        gathers[n_ch - 1].wait()
        writes[n_ch - 1] = pltpu.async_copy(
            bufs[(n_ch - 1) % 2],
            o_hbm.at[pl.ds(base + (n_ch - 1) * SC_CH, SC_CH)],
            wsems[(n_ch - 1) % 2])
        if n_ch >= 2:
            writes[n_ch - 2].wait()
        writes[n_ch - 1].wait()

    return k(w_pad, idx)


def kernel(inputs, weight):
    w_pad = jnp.concatenate(
        [weight, jnp.zeros((N_EMB, 128 - DIM), jnp.float32)], axis=1)
    xt = inputs.T
    quantized = jnp.zeros((N_TOK, DIM), jnp.float32)
    idx_parts, loss_parts = [], []
    for k in range(NCHUNK):
        idx3d, lsum = _tc_argmin_loss(xt, weight, k)
        idx = idx3d.reshape(CHT)
        q_raw = _sc_gather(w_pad, idx)
        quantized = jax.lax.dynamic_update_slice(
            quantized, q_raw[:, :DIM], (k * CHT, 0))
        idx_parts.append(idx)
        loss_parts.append(lsum[0, 0])
    loss = sum(loss_parts) * (1.25 / (N_TOK * DIM))
    indices = jnp.concatenate(idx_parts, axis=0)
    return loss, quantized, indices


# speedup vs baseline: 1.6215x; 1.1429x over previous
import jax
import jax.numpy as jnp
from jax.experimental import pallas as pl
from jax.experimental.pallas import tpu as pltpu
from jax.experimental.pallas import tpu_sc as plsc

N_TOK = 65536
N_EMB = 1024
DIM = 64
BLK = 1024
NCHUNK = 2
CHT = N_TOK // NCHUNK

SC_NC = 2
SC_NS = 16
SC_NW = SC_NC * SC_NS
SC_CH = 128


def _tc_body(xt_ref, w_ref, idx_ref, loss_ref, acc_ref):
    i = pl.program_id(0)
    xt = xt_ref[...]
    w = w_ref[...]
    c = jax.lax.dot_general(w, xt, (((1,), (0,)), ((), ())),
                            preferred_element_type=jnp.float32)
    a = jnp.sum(xt * xt, axis=0, keepdims=True)
    b = jnp.sum(w * w, axis=1, keepdims=True)
    dist = (b + a) - 2.0 * c
    m = jnp.min(dist, axis=0, keepdims=True)
    jidx = jax.lax.broadcasted_iota(
        jnp.int32, dist.shape, 0).astype(jnp.float32)
    idxf = jnp.min(jnp.where(dist == m, jidx, float(N_EMB)), axis=0)
    idx_ref[...] = idxf.astype(jnp.int32).reshape(1, 1, BLK)

    @pl.when(i == 0)
    def _():
        acc_ref[0] = 0.0

    acc_ref[0] += jnp.sum(m)

    @pl.when(i == pl.num_programs(0) - 1)
    def _():
        loss_ref[...] = jnp.full((1, 1), acc_ref[0], dtype=jnp.float32)


def _tc_argmin_loss(xt, weight, k):
    g = CHT // BLK
    return pl.pallas_call(
        _tc_body,
        grid=(g,),
        in_specs=[
            pl.BlockSpec((DIM, BLK), lambda i, k=k: (0, i + k * g)),
            pl.BlockSpec((N_EMB, DIM), lambda i: (0, 0)),
        ],
        out_specs=[
            pl.BlockSpec((1, 1, BLK), lambda i: (i, 0, 0)),
            pl.BlockSpec((1, 1), lambda i: (0, 0)),
        ],
        out_shape=[
            jax.ShapeDtypeStruct((g, 1, BLK), jnp.int32),
            jax.ShapeDtypeStruct((1, 1), jnp.float32),
        ],
        scratch_shapes=[pltpu.SMEM((1,), jnp.float32)],
    )(xt, weight)


def _sc_gather(w_pad, idx):
    n = idx.shape[0]
    rows_per_w = n // SC_NW
    n_ch = rows_per_w // SC_CH
    mesh = plsc.VectorSubcoreMesh(core_axis_name="c", subcore_axis_name="s")

    @pl.kernel(out_type=jax.ShapeDtypeStruct((n, 128), jnp.float32),
               mesh=mesh,
               scratch_types=[
                   pltpu.VMEM((rows_per_w,), jnp.int32),
                   pltpu.VMEM((SC_CH, 128), jnp.float32),
                   pltpu.VMEM((SC_CH, 128), jnp.float32),
                   pltpu.SemaphoreType.DMA,
                   pltpu.SemaphoreType.DMA,
                   pltpu.SemaphoreType.DMA,
                   pltpu.SemaphoreType.DMA,
               ])
    def k(w_hbm, i_hbm, o_hbm, idx_all, buf0, buf1, sg0, sg1, sw0, sw1):
        wid = jax.lax.axis_index("s") * SC_NC + jax.lax.axis_index("c")
        base = wid * rows_per_w
        pltpu.sync_copy(i_hbm.at[pl.ds(base, rows_per_w)], idx_all)
        bufs = (buf0, buf1)
        gsems = (sg0, sg1)
        wsems = (sw0, sw1)
        gathers = [None] * n_ch
        writes = [None] * n_ch
        for c in range(n_ch):
            s = c % 2
            if c >= 2:
                writes[c - 2].wait()
            gathers[c] = pltpu.async_copy(
                w_hbm.at[idx_all.at[pl.ds(c * SC_CH, SC_CH)]], bufs[s],
                gsems[s])
            if c >= 1:
                gathers[c - 1].wait()
                writes[c - 1] = pltpu.async_copy(
                    bufs[(c - 1) % 2],
                    o_hbm.at[pl.ds(base + (c - 1) * SC_CH, SC_CH)],
                    wsems[(c - 1) % 2])
        gathers[n_ch - 1].wait()
        writes[n_ch - 1] = pltpu.async_copy(
            bufs[(n_ch - 1) % 2],
            o_hbm.at[pl.ds(base + (n_ch - 1) * SC_CH, SC_CH)],
            wsems[(n_ch - 1) % 2])
        if n_ch >= 2:
            writes[n_ch - 2].wait()
        writes[n_ch - 1].wait()

    return k(w_pad, idx)


SL_BLK = 4096


def _packT_body(_, q_ref, o_ref):
    o_ref[...] = q_ref[:, :DIM].T


def _packT_first(q_raw):
    g = CHT // SL_BLK
    return pl.pallas_call(
        lambda q, o: _packT_body(None, q, o),
        grid=(g,),
        in_specs=[pl.BlockSpec((SL_BLK, 128), lambda i: (i, 0))],
        out_specs=pl.BlockSpec((DIM, SL_BLK), lambda i: (0, i)),
        out_shape=jax.ShapeDtypeStruct((DIM, N_TOK), jnp.float32),
    )(q_raw)


def _packT_into(buf, q_raw, k):
    g = CHT // SL_BLK
    return pl.pallas_call(
        _packT_body,
        grid=(g,),
        in_specs=[
            pl.BlockSpec(memory_space=pl.ANY),
            pl.BlockSpec((SL_BLK, 128), lambda i: (i, 0)),
        ],
        out_specs=pl.BlockSpec((DIM, SL_BLK), lambda i, k=k, g=g: (0, i + k * g)),
        out_shape=jax.ShapeDtypeStruct((DIM, N_TOK), jnp.float32),
        input_output_aliases={0: 0},
    )(buf, q_raw)


def kernel(inputs, weight):
    w_pad = jnp.concatenate(
        [weight, jnp.zeros((N_EMB, 128 - DIM), jnp.float32)], axis=1)
    xt = inputs.T
    idx_parts, loss_parts, q_raws = [], [], []
    for k in range(NCHUNK):
        idx3d, lsum = _tc_argmin_loss(xt, weight, k)
        idx = idx3d.reshape(CHT)
        q_raws.append(_sc_gather(w_pad, idx))
        idx_parts.append(idx)
        loss_parts.append(lsum[0, 0])
    qt = _packT_first(q_raws[0])
    for k in range(1, NCHUNK):
        qt = _packT_into(qt, q_raws[k], k)
    quantized = qt.T
    loss = sum(loss_parts) * (1.25 / (N_TOK * DIM))
    indices = jnp.concatenate(idx_parts, axis=0)
    return loss, quantized, indices
